# Initial kernel scaffold; baseline (speedup 1.0000x reference)
import jax
import jax.numpy as jnp
from jax.experimental import pallas as pl


def kernel(x_protein, x_substrate, edge_index_pp, edge_index_ss, edges, Wp1, bp1, Ws1, bs1, W1, b1, W2, b2):
    raise NotImplementedError("write your pallas kernel here")


# speedup vs baseline: 7.7106x; 7.7106x over previous
import functools

import jax
import jax.numpy as jnp
from jax import lax
from jax.experimental import pallas as pl
from jax.experimental.pallas import tpu as pltpu
from jax.experimental.pallas import tpu_sc as plsc

N = 10000
NPAD = 10240
E = 320000
NTILES = 16
ECHUNK = 128
ECHUNKS = 160
SUP = 32
NSUP = ECHUNKS // SUP
EPAD = NTILES * ECHUNKS * ECHUNK
PADNODE = N + 8
L = 100000
NW = 32
LCHUNKS = 25
LPAD = NW * LCHUNKS * 128
D_IN = 128
H = 256
HH = 128
ROWS_PER_TILE = NPAD // NTILES

_mesh = functools.partial(
    plsc.VectorSubcoreMesh, core_axis_name="c", subcore_axis_name="s")


def _sc_deg_body(dstp, dsts, degp, degs, idx_v, ones_v, wb_v, acc_sh):
    c = lax.axis_index("c")
    s = lax.axis_index("s")

    for k in range(ECHUNK // 16):
        ones_v[pl.ds(k * 16, 16)] = jnp.ones((16,), jnp.float32)

    def zero_wb(i, _):
        wb_v[pl.ds(i * 16, 16)] = jnp.zeros((16,), jnp.float32)
        return 0
    lax.fori_loop(0, ROWS_PER_TILE // 16, zero_wb, 0)

    def run(dst_r, out_r):
        pltpu.sync_copy(dst_r.at[s], idx_v)
        pltpu.sync_copy(wb_v, acc_sh.at[pl.ds(s * ROWS_PER_TILE, ROWS_PER_TILE)])
        plsc.subcore_barrier()

        def chunk(j, carry):
            pltpu.sync_copy(ones_v, acc_sh.at[idx_v.at[j]], add=True)
            return carry
        lax.fori_loop(0, ECHUNKS, chunk, 0)
        plsc.subcore_barrier()
        pltpu.sync_copy(acc_sh.at[pl.ds(s * ROWS_PER_TILE, ROWS_PER_TILE)], wb_v)
        pltpu.sync_copy(wb_v, out_r.at[pl.ds(s * ROWS_PER_TILE, ROWS_PER_TILE)])

    @pl.when(c == 0)
    def _():
        run(dstp, degp)

    @pl.when(c == 1)
    def _():
        run(dsts, degs)


def _sc_deg(dstp, dsts):
    return pl.kernel(
        _sc_deg_body,
        out_type=[
            jax.ShapeDtypeStruct((NPAD,), jnp.float32),
            jax.ShapeDtypeStruct((NPAD,), jnp.float32),
        ],
        mesh=_mesh(),
        scratch_types=[
            pltpu.VMEM((ECHUNKS, ECHUNK), jnp.int32),
            pltpu.VMEM((ECHUNK,), jnp.float32),
            pltpu.VMEM((ROWS_PER_TILE,), jnp.float32),
            pltpu.VMEM_SHARED((NPAD,), jnp.float32),
        ],
    )(dstp, dsts)


def _sc_agg_body(srcp, dstp, srcs, dsts, gp0, gp1, gs0, gs1,
                 ap0, ap1, as0, as1, idx_s, idx_d, msg, acc_sh, sem):
    c = lax.axis_index("c")
    s = lax.axis_index("s")

    def zero_msg(i, _):
        for k in range(HH // 16):
            msg[i, pl.ds(k * 16, 16)] = jnp.zeros((16,), jnp.float32)
        return 0

    def run(src_r, dst_r, g0, g1, o0, o1):
        for g_r, o_r in ((g0, o0), (g1, o1)):
            lax.fori_loop(0, ECHUNK, zero_msg, 0)
            for k in range(ROWS_PER_TILE // ECHUNK):
                pltpu.sync_copy(
                    msg, acc_sh.at[pl.ds(s * ROWS_PER_TILE + k * ECHUNK, ECHUNK)])
            plsc.subcore_barrier()

            def chunk(j, carry):
                pltpu.async_copy(g_r.at[idx_s.at[j]], msg, sem).wait()
                pltpu.sync_copy(msg, acc_sh.at[idx_d.at[j]], add=True)
                return carry

            for sb in range(NSUP):
                pltpu.sync_copy(src_r.at[s, pl.ds(sb * SUP, SUP)], idx_s)
                pltpu.sync_copy(dst_r.at[s, pl.ds(sb * SUP, SUP)], idx_d)
                lax.fori_loop(0, SUP, chunk, 0)
            plsc.subcore_barrier()
            for k in range(ROWS_PER_TILE // ECHUNK):
                rows = pl.ds(s * ROWS_PER_TILE + k * ECHUNK, ECHUNK)
                pltpu.sync_copy(acc_sh.at[rows], msg)
                pltpu.sync_copy(msg, o_r.at[rows])
            plsc.subcore_barrier()

    @pl.when(c == 0)
    def _():
        run(srcp, dstp, gp0, gp1, ap0, ap1)

    @pl.when(c == 1)
    def _():
        run(srcs, dsts, gs0, gs1, as0, as1)


def _sc_agg(srcp, dstp, srcs, dsts, gp0, gp1, gs0, gs1):
    node_half = jax.ShapeDtypeStruct((NPAD, HH), jnp.float32)
    return pl.kernel(
        _sc_agg_body,
        out_type=[node_half] * 4,
        mesh=_mesh(),
        scratch_types=[
            pltpu.VMEM((SUP, ECHUNK), jnp.int32),
            pltpu.VMEM((SUP, ECHUNK), jnp.int32),
            pltpu.VMEM((ECHUNK, HH), jnp.float32),
            pltpu.VMEM_SHARED((NPAD, HH), jnp.float32),
            pltpu.SemaphoreType.DMA,
        ],
    )(srcp, dstp, srcs, dsts, gp0, gp1, gs0, gs1)


def _sc_pair_body(p_r, s_r, e0_r, e1_r, pg_r, sg_r, idx0, idx1, bufp, bufs, sem):
    w = lax.axis_index("c") * NTILES + lax.axis_index("s")
    pltpu.sync_copy(e0_r.at[w], idx0)
    pltpu.sync_copy(e1_r.at[w], idx1)

    def chunk(j, carry):
        rows = pl.ds(w * LCHUNKS * 128 + j * 128, 128)
        pltpu.async_copy(p_r.at[idx0.at[j]], bufp, sem).wait()
        pltpu.sync_copy(bufp, pg_r.at[rows])
        pltpu.async_copy(s_r.at[idx1.at[j]], bufs, sem).wait()
        pltpu.sync_copy(bufs, sg_r.at[rows])
        return carry
    lax.fori_loop(0, LCHUNKS, chunk, 0)


def _sc_pair(p, sarr, e0, e1):
    out = jax.ShapeDtypeStruct((LPAD, H), jnp.float32)
    return pl.kernel(
        _sc_pair_body,
        out_type=[out, out],
        mesh=_mesh(),
        scratch_types=[
            pltpu.VMEM((LCHUNKS, 128), jnp.int32),
            pltpu.VMEM((LCHUNKS, 128), jnp.int32),
            pltpu.VMEM((128, H), jnp.float32),
            pltpu.VMEM((128, H), jnp.float32),
            pltpu.SemaphoreType.DMA,
        ],
    )(p, sarr, e0, e1)


def _tc_encode_body(x_ref, w_ref, deg_ref, g0_ref, g1_ref):
    h = jnp.dot(x_ref[...], w_ref[...], preferred_element_type=jnp.float32)
    dinv = lax.rsqrt(deg_ref[...] + 1.0)
    g = h * dinv
    g0_ref[...] = g[:, :HH]
    g1_ref[...] = g[:, HH:]


def _tc_encode(x, w, deg2):
    blk = 1280
    grid = NPAD // blk
    return pl.pallas_call(
        _tc_encode_body,
        grid=(grid,),
        in_specs=[
            pl.BlockSpec((blk, D_IN), lambda i: (i, 0)),
            pl.BlockSpec((D_IN, H), lambda i: (0, 0)),
            pl.BlockSpec((blk, 1), lambda i: (i, 0)),
        ],
        out_specs=[
            pl.BlockSpec((blk, HH), lambda i: (i, 0)),
            pl.BlockSpec((blk, HH), lambda i: (i, 0)),
        ],
        out_shape=[
            jax.ShapeDtypeStruct((NPAD, HH), jnp.float32),
            jax.ShapeDtypeStruct((NPAD, HH), jnp.float32),
        ],
    )(x, w, deg2)


def _tc_finish_body(a0, a1, g0, g1, deg_ref, b_ref, w1_ref, b1_ref, p_ref):
    dinv = lax.rsqrt(deg_ref[...] + 1.0)
    z0 = dinv * (a0[...] + g0[...]) + b_ref[:, :HH]
    z1 = dinv * (a1[...] + g1[...]) + b_ref[:, HH:]
    z = jnp.concatenate([z0, z1], axis=1)
    p_ref[...] = (
        jnp.dot(z, w1_ref[...], preferred_element_type=jnp.float32) + b1_ref[...])


def _tc_finish(a0, a1, g0, g1, deg2, b2d, w1h, b12d):
    blk = 1280
    grid = NPAD // blk
    return pl.pallas_call(
        _tc_finish_body,
        grid=(grid,),
        in_specs=[
            pl.BlockSpec((blk, HH), lambda i: (i, 0)),
            pl.BlockSpec((blk, HH), lambda i: (i, 0)),
            pl.BlockSpec((blk, HH), lambda i: (i, 0)),
            pl.BlockSpec((blk, HH), lambda i: (i, 0)),
            pl.BlockSpec((blk, 1), lambda i: (i, 0)),
            pl.BlockSpec((1, H), lambda i: (0, 0)),
            pl.BlockSpec((H, H), lambda i: (0, 0)),
            pl.BlockSpec((1, H), lambda i: (0, 0)),
        ],
        out_specs=pl.BlockSpec((blk, H), lambda i: (i, 0)),
        out_shape=jax.ShapeDtypeStruct((NPAD, H), jnp.float32),
    )(a0, a1, g0, g1, deg2, b2d, w1h, b12d)


def _tc_head_body(pg_ref, sg_ref, w2_ref, b2_ref, o_ref):
    hid = jnp.maximum(pg_ref[...] + sg_ref[...], 0.0)
    o_ref[...] = (
        jnp.dot(hid, w2_ref[...], preferred_element_type=jnp.float32)
        + b2_ref[...])


def _tc_head(pg, sg, w2, b22d):
    blk = 2048
    grid = LPAD // blk
    return pl.pallas_call(
        _tc_head_body,
        grid=(grid,),
        in_specs=[
            pl.BlockSpec((blk, H), lambda i: (i, 0)),
            pl.BlockSpec((blk, H), lambda i: (i, 0)),
            pl.BlockSpec((H, 1), lambda i: (0, 0)),
            pl.BlockSpec((1, 1), lambda i: (0, 0)),
        ],
        out_specs=pl.BlockSpec((blk, 1), lambda i: (i, 0)),
        out_shape=jax.ShapeDtypeStruct((LPAD, 1), jnp.float32),
    )(pg, sg, w2, b22d)


def _pad_edges(v):
    v = jnp.concatenate([v, jnp.full((EPAD - E,), PADNODE, jnp.int32)])
    return v.reshape(NTILES, ECHUNKS, ECHUNK)


def kernel(x_protein, x_substrate, edge_index_pp, edge_index_ss, edges,
           Wp1, bp1, Ws1, bs1, W1, b1, W2, b2):
    srcp = _pad_edges(edge_index_pp[0])
    dstp = _pad_edges(edge_index_pp[1])
    srcs = _pad_edges(edge_index_ss[0])
    dsts = _pad_edges(edge_index_ss[1])
    e0 = jnp.concatenate([edges[0], jnp.zeros((LPAD - L,), jnp.int32)])
    e1 = jnp.concatenate([edges[1], jnp.zeros((LPAD - L,), jnp.int32)])
    e0 = e0.reshape(NW, LCHUNKS, 128)
    e1 = e1.reshape(NW, LCHUNKS, 128)
    xp = jnp.pad(x_protein, ((0, NPAD - N), (0, 0)))
    xs = jnp.pad(x_substrate, ((0, NPAD - N), (0, 0)))

    degp, degs = _sc_deg(dstp, dsts)
    degp2 = degp.reshape(NPAD, 1)
    degs2 = degs.reshape(NPAD, 1)

    gp0, gp1 = _tc_encode(xp, Wp1, degp2)
    gs0, gs1 = _tc_encode(xs, Ws1, degs2)

    ap0, ap1, as0, as1 = _sc_agg(srcp, dstp, srcs, dsts, gp0, gp1, gs0, gs1)

    w1p = W1[:H, :]
    w1s = W1[H:, :]
    p = _tc_finish(ap0, ap1, gp0, gp1, degp2, bp1.reshape(1, H), w1p,
                   b1.reshape(1, H))
    szz = _tc_finish(as0, as1, gs0, gs1, degs2, bs1.reshape(1, H), w1s,
                     jnp.zeros((1, H), jnp.float32))

    pg, sg = _sc_pair(p, szz, e0, e1)
    out = _tc_head(pg, sg, W2, b2.reshape(1, 1))
    return out[:L, 0]


# speedup vs baseline: 9.1487x; 1.1865x over previous
import functools

import jax
import jax.numpy as jnp
from jax import lax
from jax.experimental import pallas as pl
from jax.experimental.pallas import tpu as pltpu
from jax.experimental.pallas import tpu_sc as plsc

N = 10000
NPAD = 10240
E = 320000
NTILES = 16
ECHUNK = 128
ECHUNKS = 160
SUP = 32
NSUP = ECHUNKS // SUP
EPAD = NTILES * ECHUNKS * ECHUNK
PADNODE = N + 8
L = 100000
NW = 32
LROW = 64
LCHUNKS = 50
LPAD = NW * LCHUNKS * LROW
D_IN = 128
H = 256
HH = 128
ROWS_PER_TILE = NPAD // NTILES

_mesh = functools.partial(
    plsc.VectorSubcoreMesh, core_axis_name="c", subcore_axis_name="s")


def _sc_deg_body(dstp, dsts, degp, degs, idx_v, ones_v, wb_v, acc_sh, sem):
    c = lax.axis_index("c")
    s = lax.axis_index("s")

    for k in range(ECHUNK // 16):
        ones_v[pl.ds(k * 16, 16)] = jnp.ones((16,), jnp.float32)

    def zero_wb(i, _):
        wb_v[pl.ds(i * 16, 16)] = jnp.zeros((16,), jnp.float32)
        return 0
    lax.fori_loop(0, ROWS_PER_TILE // 16, zero_wb, 0)

    def run(dst_r, out_r):
        pltpu.sync_copy(dst_r.at[s], idx_v)
        pltpu.sync_copy(wb_v, acc_sh.at[pl.ds(s * ROWS_PER_TILE, ROWS_PER_TILE)])
        plsc.subcore_barrier()

        def blk(i, carry):
            cps = [
                pltpu.async_copy(
                    ones_v, acc_sh.at[idx_v.at[i * 8 + b]], sem, add=True)
                for b in range(8)
            ]
            for cp in cps:
                cp.wait()
            return carry
        lax.fori_loop(0, ECHUNKS // 8, blk, 0)
        plsc.subcore_barrier()
        pltpu.sync_copy(acc_sh.at[pl.ds(s * ROWS_PER_TILE, ROWS_PER_TILE)], wb_v)
        pltpu.sync_copy(wb_v, out_r.at[pl.ds(s * ROWS_PER_TILE, ROWS_PER_TILE)])

    @pl.when(c == 0)
    def _():
        run(dstp, degp)

    @pl.when(c == 1)
    def _():
        run(dsts, degs)


def _sc_deg(dstp, dsts):
    return pl.kernel(
        _sc_deg_body,
        out_type=[
            jax.ShapeDtypeStruct((NPAD,), jnp.float32),
            jax.ShapeDtypeStruct((NPAD,), jnp.float32),
        ],
        mesh=_mesh(),
        scratch_types=[
            pltpu.VMEM((ECHUNKS, ECHUNK), jnp.int32),
            pltpu.VMEM((ECHUNK,), jnp.float32),
            pltpu.VMEM((ROWS_PER_TILE,), jnp.float32),
            pltpu.VMEM_SHARED((NPAD,), jnp.float32),
            pltpu.SemaphoreType.DMA,
        ],
    )(dstp, dsts)


def _sc_agg_body(srcp, dstp, srcs, dsts, gp0, gp1, gs0, gs1,
                 ap0, ap1, as0, as1, idx_s, idx_d, msg0, msg1, acc_sh,
                 semg, sems):
    c = lax.axis_index("c")
    s = lax.axis_index("s")

    def zero_msg(i, _):
        for k in range(HH // 16):
            msg0[i, pl.ds(k * 16, 16)] = jnp.zeros((16,), jnp.float32)
        return 0

    def run(src_r, dst_r, g0, g1, o0, o1):
        for g_r, o_r in ((g0, o0), (g1, o1)):
            lax.fori_loop(0, ECHUNK, zero_msg, 0)
            for k in range(ROWS_PER_TILE // ECHUNK):
                pltpu.sync_copy(
                    msg0, acc_sh.at[pl.ds(s * ROWS_PER_TILE + k * ECHUNK, ECHUNK)])
            plsc.subcore_barrier()

            def pair(i, carry):
                cg0 = pltpu.async_copy(g_r.at[idx_s.at[2 * i]], msg0, semg)
                cg1 = pltpu.async_copy(g_r.at[idx_s.at[2 * i + 1]], msg1, semg)
                cg0.wait()
                cs0 = pltpu.async_copy(
                    msg0, acc_sh.at[idx_d.at[2 * i]], sems, add=True)
                cg1.wait()
                cs1 = pltpu.async_copy(
                    msg1, acc_sh.at[idx_d.at[2 * i + 1]], sems, add=True)
                cs0.wait()
                cs1.wait()
                return carry

            for sb in range(NSUP):
                pltpu.sync_copy(src_r.at[s, pl.ds(sb * SUP, SUP)], idx_s)
                pltpu.sync_copy(dst_r.at[s, pl.ds(sb * SUP, SUP)], idx_d)
                lax.fori_loop(0, SUP // 2, pair, 0)
            plsc.subcore_barrier()
            for k in range(ROWS_PER_TILE // ECHUNK):
                rows = pl.ds(s * ROWS_PER_TILE + k * ECHUNK, ECHUNK)
                pltpu.sync_copy(acc_sh.at[rows], msg0)
                pltpu.sync_copy(msg0, o_r.at[rows])
            plsc.subcore_barrier()

    @pl.when(c == 0)
    def _():
        run(srcp, dstp, gp0, gp1, ap0, ap1)

    @pl.when(c == 1)
    def _():
        run(srcs, dsts, gs0, gs1, as0, as1)


def _sc_agg(srcp, dstp, srcs, dsts, gp0, gp1, gs0, gs1):
    node_half = jax.ShapeDtypeStruct((NPAD, HH), jnp.float32)
    return pl.kernel(
        _sc_agg_body,
        out_type=[node_half] * 4,
        mesh=_mesh(),
        scratch_types=[
            pltpu.VMEM((SUP, ECHUNK), jnp.int32),
            pltpu.VMEM((SUP, ECHUNK), jnp.int32),
            pltpu.VMEM((ECHUNK, HH), jnp.float32),
            pltpu.VMEM((ECHUNK, HH), jnp.float32),
            pltpu.VMEM_SHARED((NPAD, HH), jnp.float32),
            pltpu.SemaphoreType.DMA,
            pltpu.SemaphoreType.DMA,
        ],
    )(srcp, dstp, srcs, dsts, gp0, gp1, gs0, gs1)


def _sc_pair_body(p_r, s_r, e0_r, e1_r, pg_r, sg_r, idx0, idx1,
                  bufp0, bufp1, bufs0, bufs1, semg, semw):
    w = lax.axis_index("c") * NTILES + lax.axis_index("s")
    pltpu.sync_copy(e0_r.at[w], idx0)
    pltpu.sync_copy(e1_r.at[w], idx1)

    def blk(i, carry):
        rows0 = pl.ds(w * LCHUNKS * LROW + (2 * i) * LROW, LROW)
        rows1 = pl.ds(w * LCHUNKS * LROW + (2 * i + 1) * LROW, LROW)
        c0 = pltpu.async_copy(p_r.at[idx0.at[2 * i]], bufp0, semg)
        c1 = pltpu.async_copy(s_r.at[idx1.at[2 * i]], bufs0, semg)
        c2 = pltpu.async_copy(p_r.at[idx0.at[2 * i + 1]], bufp1, semg)
        c3 = pltpu.async_copy(s_r.at[idx1.at[2 * i + 1]], bufs1, semg)
        c0.wait()
        w0 = pltpu.async_copy(bufp0, pg_r.at[rows0], semw)
        c1.wait()
        w1 = pltpu.async_copy(bufs0, sg_r.at[rows0], semw)
        c2.wait()
        w2 = pltpu.async_copy(bufp1, pg_r.at[rows1], semw)
        c3.wait()
        w3 = pltpu.async_copy(bufs1, sg_r.at[rows1], semw)
        for cp in (w0, w1, w2, w3):
            cp.wait()
        return carry
    lax.fori_loop(0, LCHUNKS // 2, blk, 0)


def _sc_pair(p, sarr, e0, e1):
    out = jax.ShapeDtypeStruct((LPAD, H), jnp.float32)
    buf = pltpu.VMEM((LROW, H), jnp.float32)
    return pl.kernel(
        _sc_pair_body,
        out_type=[out, out],
        mesh=_mesh(),
        scratch_types=[
            pltpu.VMEM((LCHUNKS, LROW), jnp.int32),
            pltpu.VMEM((LCHUNKS, LROW), jnp.int32),
            buf, buf, buf, buf,
            pltpu.SemaphoreType.DMA,
            pltpu.SemaphoreType.DMA,
        ],
    )(p, sarr, e0, e1)


def _tc_encode_body(x_ref, w_ref, deg_ref, g0_ref, g1_ref):
    h = jnp.dot(x_ref[...], w_ref[...], preferred_element_type=jnp.float32)
    dinv = lax.rsqrt(deg_ref[...] + 1.0)
    g = h * dinv
    g0_ref[...] = g[:, :HH]
    g1_ref[...] = g[:, HH:]


def _tc_encode(x, w, deg2):
    blk = 1280
    grid = NPAD // blk
    return pl.pallas_call(
        _tc_encode_body,
        grid=(grid,),
        in_specs=[
            pl.BlockSpec((blk, D_IN), lambda i: (i, 0)),
            pl.BlockSpec((D_IN, H), lambda i: (0, 0)),
            pl.BlockSpec((blk, 1), lambda i: (i, 0)),
        ],
        out_specs=[
            pl.BlockSpec((blk, HH), lambda i: (i, 0)),
            pl.BlockSpec((blk, HH), lambda i: (i, 0)),
        ],
        out_shape=[
            jax.ShapeDtypeStruct((NPAD, HH), jnp.float32),
            jax.ShapeDtypeStruct((NPAD, HH), jnp.float32),
        ],
    )(x, w, deg2)


def _tc_finish_body(a0, a1, g0, g1, deg_ref, b_ref, w1_ref, b1_ref, p_ref):
    dinv = lax.rsqrt(deg_ref[...] + 1.0)
    z0 = dinv * (a0[...] + g0[...]) + b_ref[:, :HH]
    z1 = dinv * (a1[...] + g1[...]) + b_ref[:, HH:]
    z = jnp.concatenate([z0, z1], axis=1)
    p_ref[...] = (
        jnp.dot(z, w1_ref[...], preferred_element_type=jnp.float32) + b1_ref[...])


def _tc_finish(a0, a1, g0, g1, deg2, b2d, w1h, b12d):
    blk = 1280
    grid = NPAD // blk
    return pl.pallas_call(
        _tc_finish_body,
        grid=(grid,),
        in_specs=[
            pl.BlockSpec((blk, HH), lambda i: (i, 0)),
            pl.BlockSpec((blk, HH), lambda i: (i, 0)),
            pl.BlockSpec((blk, HH), lambda i: (i, 0)),
            pl.BlockSpec((blk, HH), lambda i: (i, 0)),
            pl.BlockSpec((blk, 1), lambda i: (i, 0)),
            pl.BlockSpec((1, H), lambda i: (0, 0)),
            pl.BlockSpec((H, H), lambda i: (0, 0)),
            pl.BlockSpec((1, H), lambda i: (0, 0)),
        ],
        out_specs=pl.BlockSpec((blk, H), lambda i: (i, 0)),
        out_shape=jax.ShapeDtypeStruct((NPAD, H), jnp.float32),
    )(a0, a1, g0, g1, deg2, b2d, w1h, b12d)


def _tc_head_body(pg_ref, sg_ref, w2_ref, b2_ref, o_ref):
    hid = jnp.maximum(pg_ref[...] + sg_ref[...], 0.0)
    o_ref[...] = (
        jnp.dot(hid, w2_ref[...], preferred_element_type=jnp.float32)
        + b2_ref[...])


def _tc_head(pg, sg, w2, b22d):
    blk = 2048
    grid = LPAD // blk
    return pl.pallas_call(
        _tc_head_body,
        grid=(grid,),
        in_specs=[
            pl.BlockSpec((blk, H), lambda i: (i, 0)),
            pl.BlockSpec((blk, H), lambda i: (i, 0)),
            pl.BlockSpec((H, 1), lambda i: (0, 0)),
            pl.BlockSpec((1, 1), lambda i: (0, 0)),
        ],
        out_specs=pl.BlockSpec((blk, 1), lambda i: (i, 0)),
        out_shape=jax.ShapeDtypeStruct((LPAD, 1), jnp.float32),
    )(pg, sg, w2, b22d)


def _pad_edges(v):
    v = jnp.concatenate([v, jnp.full((EPAD - E,), PADNODE, jnp.int32)])
    return v.reshape(NTILES, ECHUNKS, ECHUNK)


def kernel(x_protein, x_substrate, edge_index_pp, edge_index_ss, edges,
           Wp1, bp1, Ws1, bs1, W1, b1, W2, b2):
    srcp = _pad_edges(edge_index_pp[0])
    dstp = _pad_edges(edge_index_pp[1])
    srcs = _pad_edges(edge_index_ss[0])
    dsts = _pad_edges(edge_index_ss[1])
    e0 = jnp.concatenate([edges[0], jnp.zeros((LPAD - L,), jnp.int32)])
    e1 = jnp.concatenate([edges[1], jnp.zeros((LPAD - L,), jnp.int32)])
    e0 = e0.reshape(NW, LCHUNKS, LROW)
    e1 = e1.reshape(NW, LCHUNKS, LROW)
    xp = jnp.pad(x_protein, ((0, NPAD - N), (0, 0)))
    xs = jnp.pad(x_substrate, ((0, NPAD - N), (0, 0)))

    degp, degs = _sc_deg(dstp, dsts)
    degp2 = degp.reshape(NPAD, 1)
    degs2 = degs.reshape(NPAD, 1)

    gp0, gp1 = _tc_encode(xp, Wp1, degp2)
    gs0, gs1 = _tc_encode(xs, Ws1, degs2)

    ap0, ap1, as0, as1 = _sc_agg(srcp, dstp, srcs, dsts, gp0, gp1, gs0, gs1)

    w1p = W1[:H, :]
    w1s = W1[H:, :]
    p = _tc_finish(ap0, ap1, gp0, gp1, degp2, bp1.reshape(1, H), w1p,
                   b1.reshape(1, H))
    szz = _tc_finish(as0, as1, gs0, gs1, degs2, bs1.reshape(1, H), w1s,
                     jnp.zeros((1, H), jnp.float32))

    pg, sg = _sc_pair(p, szz, e0, e1)
    out = _tc_head(pg, sg, W2, b2.reshape(1, 1))
    return out[:L, 0]


# speedup vs baseline: 10.0701x; 1.1007x over previous
import functools

import jax
import jax.numpy as jnp
from jax import lax
from jax.experimental import pallas as pl
from jax.experimental.pallas import tpu as pltpu
from jax.experimental.pallas import tpu_sc as plsc

N = 10000
NPAD = 10240
E = 320000
NTILES = 16
ECHUNK = 128
ECHUNKS = 160
SUP = 32
NSUP = ECHUNKS // SUP
EPAD = NTILES * ECHUNKS * ECHUNK
PADNODE = N + 8
L = 100000
NW = 32
LROW = 64
LCHUNKS = 50
LPAD = NW * LCHUNKS * LROW
D_IN = 128
H = 256
HH = 128
ROWS_PER_TILE = NPAD // NTILES

_mesh = functools.partial(
    plsc.VectorSubcoreMesh, core_axis_name="c", subcore_axis_name="s")


def _sc_deg_body(dstp, dsts, degp, degs, idx_v, ones_v, wb_v, acc_sh, sem):
    c = lax.axis_index("c")
    s = lax.axis_index("s")

    for k in range(ECHUNK // 16):
        ones_v[pl.ds(k * 16, 16)] = jnp.ones((16,), jnp.float32)

    def zero_wb(i, _):
        wb_v[pl.ds(i * 16, 16)] = jnp.zeros((16,), jnp.float32)
        return 0
    lax.fori_loop(0, ROWS_PER_TILE // 16, zero_wb, 0)

    def run(dst_r, out_r):
        pltpu.sync_copy(dst_r.at[s], idx_v)
        pltpu.sync_copy(wb_v, acc_sh.at[pl.ds(s * ROWS_PER_TILE, ROWS_PER_TILE)])
        plsc.subcore_barrier()

        def blk(i, carry):
            cps = [
                pltpu.async_copy(
                    ones_v, acc_sh.at[idx_v.at[i * 8 + b]], sem, add=True)
                for b in range(8)
            ]
            for cp in cps:
                cp.wait()
            return carry
        lax.fori_loop(0, ECHUNKS // 8, blk, 0)
        plsc.subcore_barrier()
        pltpu.sync_copy(acc_sh.at[pl.ds(s * ROWS_PER_TILE, ROWS_PER_TILE)], wb_v)
        pltpu.sync_copy(wb_v, out_r.at[pl.ds(s * ROWS_PER_TILE, ROWS_PER_TILE)])

    @pl.when(c == 0)
    def _():
        run(dstp, degp)

    @pl.when(c == 1)
    def _():
        run(dsts, degs)


def _sc_deg(dstp, dsts):
    return pl.kernel(
        _sc_deg_body,
        out_type=[
            jax.ShapeDtypeStruct((NPAD,), jnp.float32),
            jax.ShapeDtypeStruct((NPAD,), jnp.float32),
        ],
        mesh=_mesh(),
        scratch_types=[
            pltpu.VMEM((ECHUNKS, ECHUNK), jnp.int32),
            pltpu.VMEM((ECHUNK,), jnp.float32),
            pltpu.VMEM((ROWS_PER_TILE,), jnp.float32),
            pltpu.VMEM_SHARED((NPAD,), jnp.float32),
            pltpu.SemaphoreType.DMA,
        ],
    )(dstp, dsts)


def _sc_agg_body(srcp, dstp, srcs, dsts, gp0, gp1, gs0, gs1,
                 ap0, ap1, as0, as1, idx_s, idx_d, msg0, msg1, acc_sh,
                 semg, sems):
    c = lax.axis_index("c")
    s = lax.axis_index("s")

    def zero_msg(i, _):
        for k in range(HH // 16):
            msg0[i, pl.ds(k * 16, 16)] = jnp.zeros((16,), jnp.float32)
        return 0

    def run(src_r, dst_r, g0, g1, o0, o1):
        for g_r, o_r in ((g0, o0), (g1, o1)):
            lax.fori_loop(0, ECHUNK, zero_msg, 0)
            for k in range(ROWS_PER_TILE // ECHUNK):
                pltpu.sync_copy(
                    msg0, acc_sh.at[pl.ds(s * ROWS_PER_TILE + k * ECHUNK, ECHUNK)])
            plsc.subcore_barrier()

            msgs = (msg0, msg1)

            def sblock(sb, carry):
                pltpu.sync_copy(src_r.at[s, pl.ds(sb * SUP, SUP)], idx_s)
                pltpu.sync_copy(dst_r.at[s, pl.ds(sb * SUP, SUP)], idx_d)
                gd, sd = {}, {}
                for k in range(SUP + 1):
                    if k < SUP:
                        if k >= 2:
                            sd[k - 2].wait()
                        gd[k] = pltpu.async_copy(
                            g_r.at[idx_s.at[k]], msgs[k & 1], semg)
                    j = k - 1
                    if 0 <= j < SUP:
                        gd[j].wait()
                        sd[j] = pltpu.async_copy(
                            msgs[j & 1], acc_sh.at[idx_d.at[j]], sems,
                            add=True)
                sd[SUP - 2].wait()
                sd[SUP - 1].wait()
                return carry

            lax.fori_loop(0, NSUP, sblock, 0)
            plsc.subcore_barrier()
            for k in range(ROWS_PER_TILE // ECHUNK):
                rows = pl.ds(s * ROWS_PER_TILE + k * ECHUNK, ECHUNK)
                pltpu.sync_copy(acc_sh.at[rows], msg0)
                pltpu.sync_copy(msg0, o_r.at[rows])
            plsc.subcore_barrier()

    @pl.when(c == 0)
    def _():
        run(srcp, dstp, gp0, gp1, ap0, ap1)

    @pl.when(c == 1)
    def _():
        run(srcs, dsts, gs0, gs1, as0, as1)


def _sc_agg(srcp, dstp, srcs, dsts, gp0, gp1, gs0, gs1):
    node_half = jax.ShapeDtypeStruct((NPAD, HH), jnp.float32)
    return pl.kernel(
        _sc_agg_body,
        out_type=[node_half] * 4,
        mesh=_mesh(),
        scratch_types=[
            pltpu.VMEM((SUP, ECHUNK), jnp.int32),
            pltpu.VMEM((SUP, ECHUNK), jnp.int32),
            pltpu.VMEM((ECHUNK, HH), jnp.float32),
            pltpu.VMEM((ECHUNK, HH), jnp.float32),
            pltpu.VMEM_SHARED((NPAD, HH), jnp.float32),
            pltpu.SemaphoreType.DMA,
            pltpu.SemaphoreType.DMA,
        ],
    )(srcp, dstp, srcs, dsts, gp0, gp1, gs0, gs1)


def _sc_pair_body(p_r, s_r, e0_r, e1_r, pg_r, sg_r, idx0, idx1,
                  bufp0, bufp1, bufs0, bufs1, semg, semw):
    w = lax.axis_index("c") * NTILES + lax.axis_index("s")
    pltpu.sync_copy(e0_r.at[w], idx0)
    pltpu.sync_copy(e1_r.at[w], idx1)

    bufp = (bufp0, bufp1)
    bufs = (bufs0, bufs1)
    gp, gs, wp, ws = {}, {}, {}, {}
    for k in range(LCHUNKS + 1):
        if k < LCHUNKS:
            if k >= 2:
                wp[k - 2].wait()
                ws[k - 2].wait()
            gp[k] = pltpu.async_copy(p_r.at[idx0.at[k]], bufp[k & 1], semg)
            gs[k] = pltpu.async_copy(s_r.at[idx1.at[k]], bufs[k & 1], semg)
        j = k - 1
        if 0 <= j < LCHUNKS:
            rows = pl.ds(w * LCHUNKS * LROW + j * LROW, LROW)
            gp[j].wait()
            wp[j] = pltpu.async_copy(bufp[j & 1], pg_r.at[rows], semw)
            gs[j].wait()
            ws[j] = pltpu.async_copy(bufs[j & 1], sg_r.at[rows], semw)
    wp[LCHUNKS - 2].wait()
    ws[LCHUNKS - 2].wait()
    wp[LCHUNKS - 1].wait()
    ws[LCHUNKS - 1].wait()


def _sc_pair(p, sarr, e0, e1):
    out = jax.ShapeDtypeStruct((LPAD, H), jnp.float32)
    buf = pltpu.VMEM((LROW, H), jnp.float32)
    return pl.kernel(
        _sc_pair_body,
        out_type=[out, out],
        mesh=_mesh(),
        scratch_types=[
            pltpu.VMEM((LCHUNKS, LROW), jnp.int32),
            pltpu.VMEM((LCHUNKS, LROW), jnp.int32),
            buf, buf, buf, buf,
            pltpu.SemaphoreType.DMA,
            pltpu.SemaphoreType.DMA,
        ],
    )(p, sarr, e0, e1)


def _tc_encode_body(x_ref, w_ref, deg_ref, g0_ref, g1_ref):
    h = jnp.dot(x_ref[...], w_ref[...], preferred_element_type=jnp.float32)
    dinv = lax.rsqrt(deg_ref[...] + 1.0)
    g = h * dinv
    g0_ref[...] = g[:, :HH]
    g1_ref[...] = g[:, HH:]


def _tc_encode(x, w, deg2):
    blk = 1280
    grid = NPAD // blk
    return pl.pallas_call(
        _tc_encode_body,
        grid=(grid,),
        in_specs=[
            pl.BlockSpec((blk, D_IN), lambda i: (i, 0)),
            pl.BlockSpec((D_IN, H), lambda i: (0, 0)),
            pl.BlockSpec((blk, 1), lambda i: (i, 0)),
        ],
        out_specs=[
            pl.BlockSpec((blk, HH), lambda i: (i, 0)),
            pl.BlockSpec((blk, HH), lambda i: (i, 0)),
        ],
        out_shape=[
            jax.ShapeDtypeStruct((NPAD, HH), jnp.float32),
            jax.ShapeDtypeStruct((NPAD, HH), jnp.float32),
        ],
    )(x, w, deg2)


def _tc_finish_body(a0, a1, g0, g1, deg_ref, b_ref, w1_ref, b1_ref, p_ref):
    dinv = lax.rsqrt(deg_ref[...] + 1.0)
    z0 = dinv * (a0[...] + g0[...]) + b_ref[:, :HH]
    z1 = dinv * (a1[...] + g1[...]) + b_ref[:, HH:]
    z = jnp.concatenate([z0, z1], axis=1)
    p_ref[...] = (
        jnp.dot(z, w1_ref[...], preferred_element_type=jnp.float32) + b1_ref[...])


def _tc_finish(a0, a1, g0, g1, deg2, b2d, w1h, b12d):
    blk = 1280
    grid = NPAD // blk
    return pl.pallas_call(
        _tc_finish_body,
        grid=(grid,),
        in_specs=[
            pl.BlockSpec((blk, HH), lambda i: (i, 0)),
            pl.BlockSpec((blk, HH), lambda i: (i, 0)),
            pl.BlockSpec((blk, HH), lambda i: (i, 0)),
            pl.BlockSpec((blk, HH), lambda i: (i, 0)),
            pl.BlockSpec((blk, 1), lambda i: (i, 0)),
            pl.BlockSpec((1, H), lambda i: (0, 0)),
            pl.BlockSpec((H, H), lambda i: (0, 0)),
            pl.BlockSpec((1, H), lambda i: (0, 0)),
        ],
        out_specs=pl.BlockSpec((blk, H), lambda i: (i, 0)),
        out_shape=jax.ShapeDtypeStruct((NPAD, H), jnp.float32),
    )(a0, a1, g0, g1, deg2, b2d, w1h, b12d)


def _tc_head_body(pg_ref, sg_ref, w2_ref, b2_ref, o_ref):
    hid = jnp.maximum(pg_ref[...] + sg_ref[...], 0.0)
    o_ref[...] = (
        jnp.dot(hid, w2_ref[...], preferred_element_type=jnp.float32)
        + b2_ref[...])


def _tc_head(pg, sg, w2, b22d):
    blk = 2048
    grid = LPAD // blk
    return pl.pallas_call(
        _tc_head_body,
        grid=(grid,),
        in_specs=[
            pl.BlockSpec((blk, H), lambda i: (i, 0)),
            pl.BlockSpec((blk, H), lambda i: (i, 0)),
            pl.BlockSpec((H, 1), lambda i: (0, 0)),
            pl.BlockSpec((1, 1), lambda i: (0, 0)),
        ],
        out_specs=pl.BlockSpec((blk, 1), lambda i: (i, 0)),
        out_shape=jax.ShapeDtypeStruct((LPAD, 1), jnp.float32),
    )(pg, sg, w2, b22d)


def _pad_edges(v):
    v = jnp.concatenate([v, jnp.full((EPAD - E,), PADNODE, jnp.int32)])
    return v.reshape(NTILES, ECHUNKS, ECHUNK)


def kernel(x_protein, x_substrate, edge_index_pp, edge_index_ss, edges,
           Wp1, bp1, Ws1, bs1, W1, b1, W2, b2):
    srcp = _pad_edges(edge_index_pp[0])
    dstp = _pad_edges(edge_index_pp[1])
    srcs = _pad_edges(edge_index_ss[0])
    dsts = _pad_edges(edge_index_ss[1])
    e0 = jnp.concatenate([edges[0], jnp.zeros((LPAD - L,), jnp.int32)])
    e1 = jnp.concatenate([edges[1], jnp.zeros((LPAD - L,), jnp.int32)])
    e0 = e0.reshape(NW, LCHUNKS, LROW)
    e1 = e1.reshape(NW, LCHUNKS, LROW)
    xp = jnp.pad(x_protein, ((0, NPAD - N), (0, 0)))
    xs = jnp.pad(x_substrate, ((0, NPAD - N), (0, 0)))

    degp, degs = _sc_deg(dstp, dsts)
    degp2 = degp.reshape(NPAD, 1)
    degs2 = degs.reshape(NPAD, 1)

    gp0, gp1 = _tc_encode(xp, Wp1, degp2)
    gs0, gs1 = _tc_encode(xs, Ws1, degs2)

    ap0, ap1, as0, as1 = _sc_agg(srcp, dstp, srcs, dsts, gp0, gp1, gs0, gs1)

    w1p = W1[:H, :]
    w1s = W1[H:, :]
    p = _tc_finish(ap0, ap1, gp0, gp1, degp2, bp1.reshape(1, H), w1p,
                   b1.reshape(1, H))
    szz = _tc_finish(as0, as1, gs0, gs1, degs2, bs1.reshape(1, H), w1s,
                     jnp.zeros((1, H), jnp.float32))

    pg, sg = _sc_pair(p, szz, e0, e1)
    out = _tc_head(pg, sg, W2, b2.reshape(1, 1))
    return out[:L, 0]
